# Initial kernel scaffold; baseline (speedup 1.0000x reference)
import functools

import jax
import jax.numpy as jnp
from jax import lax
from jax.experimental import pallas as pl
from jax.experimental.pallas import tpu as pltpu
from jax.experimental.pallas import tpu_sc as plsc

D = 64
B = 16384 * 50
NC, NS = 2, 16
NW = NC * NS
B_PER_W = B // NW
CHUNK = 512
N_CHUNKS = B_PER_W // CHUNK

_mesh = plsc.VectorSubcoreMesh(core_axis_name="c", subcore_axis_name="s")


@functools.partial(
    pl.kernel,
    mesh=_mesh,
    out_type=jax.ShapeDtypeStruct((B, D), jnp.float32),
    scratch_types=[
        pltpu.VMEM((CHUNK,), jnp.int32),
        pltpu.VMEM((CHUNK, D), jnp.float32),
        pltpu.SemaphoreType.DMA,
    ],
)
def _gather(table_hbm, idx_hbm, out_hbm, idx_v, rows_v, sem):
    wid = lax.axis_index("s") * NC + lax.axis_index("c")
    base = wid * B_PER_W

    def body(i, carry):
        off = base + i * CHUNK
        pltpu.sync_copy(idx_hbm.at[pl.ds(off, CHUNK)], idx_v)
        pltpu.async_copy(table_hbm.at[idx_v], rows_v, sem).wait()
        pltpu.sync_copy(rows_v, out_hbm.at[pl.ds(off, CHUNK)])
        return carry

    lax.fori_loop(0, N_CHUNKS, body, 0)


def kernel(idx, table):
    out = _gather(table, idx.reshape(-1))
    return out.reshape(idx.shape + (D,))


# speedup vs baseline: 1.7953x; 1.7953x over previous
import functools

import jax
import jax.numpy as jnp
from jax import lax
from jax.experimental import pallas as pl
from jax.experimental.pallas import tpu as pltpu
from jax.experimental.pallas import tpu_sc as plsc

D = 64
B = 16384 * 50
NC, NS = 2, 16
NW = NC * NS
B_PER_W = B // NW
CHUNK = 512
N_CHUNKS = B_PER_W // CHUNK

_mesh = plsc.VectorSubcoreMesh(core_axis_name="c", subcore_axis_name="s")


@functools.partial(
    pl.kernel,
    mesh=_mesh,
    compiler_params=pltpu.CompilerParams(use_tc_tiling_on_sc=False),
    out_type=jax.ShapeDtypeStruct((B, D), jnp.float32),
    scratch_types=[
        pltpu.VMEM((CHUNK,), jnp.int32),
        pltpu.VMEM((CHUNK, D), jnp.float32),
        pltpu.SemaphoreType.DMA,
    ],
)
def _gather(table_hbm, idx_hbm, out_hbm, idx_v, rows_v, sem):
    wid = lax.axis_index("s") * NC + lax.axis_index("c")
    base = wid * B_PER_W

    def body(i, carry):
        off = base + i * CHUNK
        pltpu.sync_copy(idx_hbm.at[pl.ds(off, CHUNK)], idx_v)
        pltpu.async_copy(table_hbm.at[idx_v], rows_v, sem).wait()
        pltpu.sync_copy(rows_v, out_hbm.at[pl.ds(off, CHUNK)])
        return carry

    lax.fori_loop(0, N_CHUNKS, body, 0)


def kernel(idx, table):
    out = _gather(table, idx.reshape(-1))
    return out.reshape(idx.shape + (D,))


# speedup vs baseline: 1.8742x; 1.0440x over previous
import functools

import jax
import jax.numpy as jnp
from jax import lax
from jax.experimental import pallas as pl
from jax.experimental.pallas import tpu as pltpu
from jax.experimental.pallas import tpu_sc as plsc

D = 64
B = 16384 * 50
NC, NS = 2, 16
NW = NC * NS
B_PER_W = B // NW
CHUNK = 400
N_CHUNKS = B_PER_W // CHUNK
NB = 4
FL = NB // 2
G_OUTER = N_CHUNKS // NB

_mesh = plsc.VectorSubcoreMesh(core_axis_name="c", subcore_axis_name="s")


@functools.partial(
    pl.kernel,
    mesh=_mesh,
    compiler_params=pltpu.CompilerParams(use_tc_tiling_on_sc=False),
    out_type=jax.ShapeDtypeStruct((B, D), jnp.float32),
    scratch_types=[
        pltpu.VMEM((B_PER_W,), jnp.int32),
        pltpu.VMEM((NB, CHUNK, D), jnp.float32),
        pltpu.SemaphoreType.DMA,
        pltpu.SemaphoreType.DMA,
        pltpu.SemaphoreType.DMA,
        pltpu.SemaphoreType.DMA,
        pltpu.SemaphoreType.DMA,
        pltpu.SemaphoreType.DMA,
        pltpu.SemaphoreType.DMA,
        pltpu.SemaphoreType.DMA,
    ],
)
def _gather(table_hbm, idx_hbm, out_hbm, idx_v, rows_v,
            sg0, sg1, sg2, sg3, so0, so1, so2, so3):
    sem_g = [sg0, sg1, sg2, sg3]
    sem_o = [so0, so1, so2, so3]
    wid = lax.axis_index("s") * NC + lax.axis_index("c")
    base = wid * B_PER_W

    pltpu.sync_copy(idx_hbm.at[pl.ds(base, B_PER_W)], idx_v)

    def g_start(i, b):
        idx_slice = idx_v.at[pl.ds(i * CHUNK, CHUNK)]
        pltpu.async_copy(table_hbm.at[idx_slice], rows_v.at[b], sem_g[b])

    def g_wait(i, b):
        idx_slice = idx_v.at[pl.ds(i * CHUNK, CHUNK)]
        pltpu.make_async_copy(table_hbm.at[idx_slice], rows_v.at[b], sem_g[b]).wait()

    def o_start(i, b):
        pltpu.async_copy(rows_v.at[b], out_hbm.at[pl.ds(base + i * CHUNK, CHUNK)],
                         sem_o[b])

    def o_wait(i, b):
        pltpu.make_async_copy(rows_v.at[b], out_hbm.at[pl.ds(base + i * CHUNK, CHUNK)],
                              sem_o[b]).wait()

    for b in range(FL):
        g_start(b, b)

    for b in range(NB):
        g_wait(b, b)
        o_start(b, b)
        j = b + FL
        bj = j % NB
        if j >= NB:
            o_wait(j - NB, bj)
        g_start(j, bj)

    def body(g, carry):
        i0 = g * NB
        for b in range(NB):
            i = i0 + b
            g_wait(i, b)
            o_start(i, b)
            j = i + FL
            bj = (b + FL) % NB
            o_wait(j - NB, bj)
            g_start(j, bj)
        return carry

    lax.fori_loop(1, G_OUTER - 1, body, 0)

    i0 = (G_OUTER - 1) * NB
    for b in range(NB):
        i = i0 + b
        g_wait(i, b)
        o_start(i, b)
        j = i + FL
        if j < N_CHUNKS:
            bj = (b + FL) % NB
            o_wait(j - NB, bj)
            g_start(j, bj)

    for b in range(NB):
        o_wait(N_CHUNKS - NB + b, b)


def kernel(idx, table):
    out = _gather(table, idx.reshape(-1))
    return out.reshape(idx.shape + (D,))
